# Initial kernel scaffold; baseline (speedup 1.0000x reference)
import jax
import jax.numpy as jnp
from jax.experimental import pallas as pl


def kernel(x, W1, b1, W2, b2, W3, b3, Wc, bc):
    raise NotImplementedError("write your pallas kernel here")


# speedup vs baseline: 6.1033x; 6.1033x over previous
import jax
import jax.numpy as jnp
from jax.experimental import pallas as pl


def _gcn_body(x_ref, w1_ref, b1_ref, w2_ref, b2_ref, w3_ref, b3_ref,
              wc_ref, bc_ref, out_ref, h_ref):
    x = x_ref[...]
    norm = jnp.sqrt(jnp.sum(x * x, axis=1, keepdims=True))
    xn = x / jnp.maximum(norm, 1e-8)

    def layer(h, w, b):
        support = jnp.dot(h, w, preferred_element_type=jnp.float32)
        t = jax.lax.dot_general(xn, support, (((0,), (0,)), ((), ())),
                                preferred_element_type=jnp.float32)
        out = jnp.dot(xn, t, preferred_element_type=jnp.float32) + b
        return jnp.where(out >= 0, out, 0.25 * out)

    h = layer(x, w1_ref[...], b1_ref[...])
    h = layer(h, w2_ref[...], b2_ref[...])
    h = layer(h, w3_ref[...], b3_ref[...])
    h_ref[...] = h
    out_ref[...] = jnp.dot(h, wc_ref[...],
                           preferred_element_type=jnp.float32) + bc_ref[...]


def kernel(x, W1, b1, W2, b2, W3, b3, Wc, bc):
    n, _ = x.shape
    out, h = pl.pallas_call(
        _gcn_body,
        out_shape=(
            jax.ShapeDtypeStruct((n, Wc.shape[1]), jnp.float32),
            jax.ShapeDtypeStruct((n, W3.shape[1]), jnp.float32),
        ),
    )(x, W1, b1[0, 0][None, :], W2, b2[0, 0][None, :],
      W3, b3[0, 0][None, :], Wc, bc[None, :])
    return (out, h)
